# baseline (speedup 1.0000x reference)
import numpy as np
import jax
import jax.numpy as jnp
from jax.experimental import pallas as pl
from jax.experimental.pallas import tpu as pltpu

B = 2
S = 2048
D = 1024
H = 16
DK = 64
W = 8
EPS = 1e-6

BQ = 256
NB = S // BQ
HALO = BQ + 2 * W
_NEG = -1e30


def _fused_kernel(hs_ref, wq_ref, wk_ref, wv_ref, wf_ref, bf_ref, g_ref,
                  bt_ref, out_ref):
    t = pl.program_id(0)
    q0 = t * BQ
    st = jnp.clip(q0 - W, 0, S - HALO)
    off = st - q0

    x = jnp.concatenate(
        [hs_ref[b, pl.ds(pl.multiple_of(q0, 8), BQ), :] for b in range(B)],
        axis=0)
    xh = jnp.concatenate(
        [hs_ref[b, pl.ds(pl.multiple_of(st, 8), HALO), :] for b in range(B)],
        axis=0)
    xb = x.astype(jnp.bfloat16)
    xhb = xh.astype(jnp.bfloat16)

    qb = jnp.dot(xb, wq_ref[...],
                 preferred_element_type=jnp.float32).astype(jnp.bfloat16)
    kb = jnp.dot(xhb, wk_ref[...],
                 preferred_element_type=jnp.float32).astype(jnp.bfloat16)
    vb = jnp.dot(xhb, wv_ref[...],
                 preferred_element_type=jnp.float32).astype(jnp.bfloat16)

    ii = jax.lax.broadcasted_iota(jnp.int32, (BQ, HALO), 0)
    jj = jax.lax.broadcasted_iota(jnp.int32, (BQ, HALO), 1)
    bias = jnp.where(jnp.abs(jj + off - ii) <= W, 0.0, _NEG)
    gi = q0 + jax.lax.broadcasted_iota(jnp.int32, (BQ, 1), 0)
    n_inv = (jnp.maximum(W - gi, 0)
             + jnp.maximum(gi + W - (S - 1), 0)).astype(jnp.float32)

    ones_col = jnp.ones((HALO, 1), jnp.bfloat16)
    outs = []
    for b in range(B):
        qB = qb[b * BQ:(b + 1) * BQ]
        kB = kb[b * HALO:(b + 1) * HALO]
        vB = vb[b * HALO:(b + 1) * HALO]
        for h in range(H):
            qh = qB[:, h * DK:(h + 1) * DK]
            kh = kB[:, h * DK:(h + 1) * DK]
            vh = jnp.concatenate(
                [vB[:, h * DK:(h + 1) * DK], ones_col], axis=1)
            sh = jax.lax.dot_general(qh, kh, (((1,), (1,)), ((), ())),
                                     preferred_element_type=jnp.float32)
            e = jnp.exp(sh + bias)
            o = jax.lax.dot_general(e.astype(jnp.bfloat16), vh,
                                    (((1,), (0,)), ((), ())),
                                    preferred_element_type=jnp.float32)
            inv = 1.0 / (o[:, DK:DK + 1] + n_inv)
            outs.append(o[:, :DK] * inv)
    attn = jnp.concatenate(
        [jnp.concatenate(outs[b * H:(b + 1) * H], axis=1) for b in range(B)],
        axis=0)

    ctx = jnp.dot(attn.astype(jnp.bfloat16), wf_ref[...],
                  preferred_element_type=jnp.float32)
    ctx = ctx + bf_ref[...] + x
    mean = jnp.mean(ctx, axis=1, keepdims=True)
    cen = ctx - mean
    var = jnp.mean(cen * cen, axis=1, keepdims=True)
    y = cen * jax.lax.rsqrt(var + EPS) * g_ref[...] + bt_ref[...]
    out_ref[0] = y[:BQ]
    out_ref[1] = y[BQ:]


def kernel(hidden_states, w_qs, w_ks, w_vs, w_fc, b_fc, gamma, beta, q_k_mask):
    del q_k_mask
    full = lambda shape: pl.BlockSpec(shape, lambda t: (0,) * len(shape))
    return pl.pallas_call(
        _fused_kernel,
        grid=(NB,),
        in_specs=[
            full((B, S, D)),
            full((D, D)),
            full((D, D)),
            full((D, D)),
            full((D, D)),
            full((1, D)),
            full((1, D)),
            full((1, D)),
        ],
        out_specs=pl.BlockSpec((B, BQ, D), lambda t: (0, t, 0)),
        out_shape=jax.ShapeDtypeStruct((B, S, D), jnp.float32),
    )(hidden_states,
      (w_qs * np.float32(1.0 / np.sqrt(DK))).astype(jnp.bfloat16),
      w_ks.astype(jnp.bfloat16),
      w_vs.astype(jnp.bfloat16), w_fc.astype(jnp.bfloat16),
      b_fc.reshape(1, D), gamma.reshape(1, D), beta.reshape(1, D))


# speedup vs baseline: 1.3418x; 1.3418x over previous
import numpy as np
import jax
import jax.numpy as jnp
from jax.experimental import pallas as pl
from jax._src.pallas.core import Element
from jax.experimental.pallas import tpu as pltpu

B = 2
S = 2048
D = 1024
H = 16
DK = 64
W = 8
EPS = 1e-6

BQ = 240
NB = -(-S // BQ)
HALO = BQ + 2 * W
_NEG = -1e30


def _fused_kernel(cur_ref, halo_ref, wq_ref, wk_ref, wv_ref, wf_ref,
                  bf_ref, g_ref, bt_ref, out_ref):
    t = pl.program_id(0)
    q0 = t * BQ
    st = jnp.maximum(q0 - W, 0)
    x = cur_ref[...].reshape(B * BQ, D)
    xh = halo_ref[...].reshape(B * HALO, D)
    hrow = st + jax.lax.broadcasted_iota(jnp.int32, (B * HALO, 1), 0) % HALO
    xh = jnp.where((hrow >= 0) & (hrow < S), xh, 0.0)
    xb = x.astype(jnp.bfloat16)
    xhb = xh.astype(jnp.bfloat16)

    qb = jnp.dot(xb, wq_ref[...],
                 preferred_element_type=jnp.float32).astype(jnp.bfloat16)
    kb = jnp.dot(xhb, wk_ref[...],
                 preferred_element_type=jnp.float32).astype(jnp.bfloat16)
    vb = jnp.dot(xhb, wv_ref[...],
                 preferred_element_type=jnp.float32).astype(jnp.bfloat16)

    ii = jax.lax.broadcasted_iota(jnp.int32, (BQ, HALO), 0)
    jj = jax.lax.broadcasted_iota(jnp.int32, (BQ, HALO), 1)
    bias = jnp.where(jnp.abs(jj + (st - q0) - ii) <= W, 0.0, _NEG)
    gi = q0 + jax.lax.broadcasted_iota(jnp.int32, (BQ, 1), 0)
    n_inv = jnp.maximum(W - gi, 0).astype(jnp.float32)

    ones_col = jnp.ones((HALO, 1), jnp.bfloat16)
    for b in range(B):
        qB = qb[b * BQ:(b + 1) * BQ]
        kB = kb[b * HALO:(b + 1) * HALO]
        vB = vb[b * HALO:(b + 1) * HALO]
        outs = []
        for h in range(H):
            sl = slice(h * DK, (h + 1) * DK)
            vh = jnp.concatenate([vB[:, sl], ones_col], axis=1)
            sh = jax.lax.dot_general(qB[:, sl], kB[:, sl],
                                     (((1,), (1,)), ((), ())),
                                     preferred_element_type=jnp.float32)
            e = jnp.exp(sh + bias)
            o = jax.lax.dot_general(e.astype(jnp.bfloat16), vh,
                                    (((1,), (0,)), ((), ())),
                                    preferred_element_type=jnp.float32)
            inv = 1.0 / (o[:, DK:DK + 1] + n_inv)
            outs.append(o[:, :DK] * inv)
        attn = jnp.concatenate(outs, axis=1)
        ctx = jnp.dot(attn.astype(jnp.bfloat16), wf_ref[...],
                      preferred_element_type=jnp.float32)
        ctx = ctx + bf_ref[...] + x[b * BQ:(b + 1) * BQ]
        mean = jnp.mean(ctx, axis=1, keepdims=True)
        cen = ctx - mean
        var = jnp.mean(cen * cen, axis=1, keepdims=True)
        out_ref[b] = (cen * jax.lax.rsqrt(var + EPS) * g_ref[...]
                      + bt_ref[...])


def kernel(hidden_states, w_qs, w_ks, w_vs, w_fc, b_fc, gamma, beta, q_k_mask):
    del q_k_mask
    full = lambda shape: pl.BlockSpec(shape, lambda t: (0,) * len(shape))
    return pl.pallas_call(
        _fused_kernel,
        grid=(NB,),
        in_specs=[
            pl.BlockSpec((B, BQ, D), lambda t: (0, t, 0)),
            pl.BlockSpec((Element(B), Element(HALO, padding=(0, NB * BQ + 2 * W - S)),
                          Element(D)),
                         lambda t: (0, W * jnp.maximum(t * (BQ // W) - 1, 0),
                                    0)),
            full((D, D)),
            full((D, D)),
            full((D, D)),
            full((D, D)),
            full((1, D)),
            full((1, D)),
            full((1, D)),
        ],
        out_specs=pl.BlockSpec((B, BQ, D), lambda t: (0, t, 0)),
        out_shape=jax.ShapeDtypeStruct((B, S, D), jnp.float32),
    )(hidden_states, hidden_states,
      (w_qs * np.float32(1.0 / np.sqrt(DK))).astype(jnp.bfloat16),
      w_ks.astype(jnp.bfloat16),
      w_vs.astype(jnp.bfloat16), w_fc.astype(jnp.bfloat16),
      b_fc.reshape(1, D), gamma.reshape(1, D), beta.reshape(1, D))


# speedup vs baseline: 1.4911x; 1.1112x over previous
import numpy as np
import jax
import jax.numpy as jnp
from jax.experimental import pallas as pl
from jax._src.pallas.core import Element
from jax.experimental.pallas import tpu as pltpu

B = 2
S = 2048
D = 1024
H = 16
DK = 64
W = 8
EPS = 1e-6

BQ = 240
NB = -(-S // BQ)
HALO = BQ + 2 * W
_NEG = -1e30


def _fused_kernel(cur_ref, halo_ref, wq_ref, wk_ref, wv_ref, wf_ref,
                  bf_ref, g_ref, bt_ref, out_ref,
                  wqs_ref, wks_ref, wvs_ref, wfs_ref):
    t = pl.program_id(0)

    @pl.when(t == 0)
    def _cast_weights():
        wqs_ref[...] = (wq_ref[...]
                        * np.float32(1.0 / np.sqrt(DK))).astype(jnp.bfloat16)
        wks_ref[...] = wk_ref[...].astype(jnp.bfloat16)
        wvs_ref[...] = wv_ref[...].astype(jnp.bfloat16)
        wfs_ref[...] = wf_ref[...].astype(jnp.bfloat16)
    q0 = t * BQ
    st = jnp.maximum(q0 - W, 0)
    x = cur_ref[...].reshape(B * BQ, D)
    xh = halo_ref[...].reshape(B * HALO, D)
    hrow = st + jax.lax.broadcasted_iota(jnp.int32, (B * HALO, 1), 0) % HALO
    xh = jnp.where((hrow >= 0) & (hrow < S), xh, 0.0)
    xb = x.astype(jnp.bfloat16)
    xhb = xh.astype(jnp.bfloat16)

    qb = jnp.dot(xb, wqs_ref[...],
                 preferred_element_type=jnp.float32).astype(jnp.bfloat16)
    kb = jnp.dot(xhb, wks_ref[...],
                 preferred_element_type=jnp.float32).astype(jnp.bfloat16)
    vb = jnp.dot(xhb, wvs_ref[...],
                 preferred_element_type=jnp.float32).astype(jnp.bfloat16)

    ii = jax.lax.broadcasted_iota(jnp.int32, (BQ, HALO), 0)
    jj = jax.lax.broadcasted_iota(jnp.int32, (BQ, HALO), 1)
    bias = jnp.where(jnp.abs(jj + (st - q0) - ii) <= W, 0.0, _NEG)
    gi = q0 + jax.lax.broadcasted_iota(jnp.int32, (BQ, 1), 0)
    n_inv = jnp.maximum(W - gi, 0).astype(jnp.float32)

    ones_col = jnp.ones((HALO, 1), jnp.bfloat16)
    for b in range(B):
        qB = qb[b * BQ:(b + 1) * BQ]
        kB = kb[b * HALO:(b + 1) * HALO]
        vB = vb[b * HALO:(b + 1) * HALO]
        outs = []
        for h in range(H):
            sl = slice(h * DK, (h + 1) * DK)
            vh = jnp.concatenate([vB[:, sl], ones_col], axis=1)
            sh = jax.lax.dot_general(qB[:, sl], kB[:, sl],
                                     (((1,), (1,)), ((), ())),
                                     preferred_element_type=jnp.float32)
            e = jnp.exp(sh + bias)
            o = jax.lax.dot_general(e.astype(jnp.bfloat16), vh,
                                    (((1,), (0,)), ((), ())),
                                    preferred_element_type=jnp.float32)
            inv = 1.0 / (o[:, DK:DK + 1] + n_inv)
            outs.append(o[:, :DK] * inv)
        attn = jnp.concatenate(outs, axis=1)
        ctx = jnp.dot(attn.astype(jnp.bfloat16), wfs_ref[...],
                      preferred_element_type=jnp.float32)
        ctx = ctx + bf_ref[...] + x[b * BQ:(b + 1) * BQ]
        mean = jnp.mean(ctx, axis=1, keepdims=True)
        cen = ctx - mean
        var = jnp.mean(cen * cen, axis=1, keepdims=True)
        out_ref[b] = (cen * jax.lax.rsqrt(var + EPS) * g_ref[...]
                      + bt_ref[...])


def kernel(hidden_states, w_qs, w_ks, w_vs, w_fc, b_fc, gamma, beta, q_k_mask):
    del q_k_mask
    full = lambda shape: pl.BlockSpec(shape, lambda t: (0,) * len(shape))
    return pl.pallas_call(
        _fused_kernel,
        grid=(NB,),
        in_specs=[
            pl.BlockSpec((B, BQ, D), lambda t: (0, t, 0)),
            pl.BlockSpec((Element(B), Element(HALO, padding=(0, NB * BQ + 2 * W - S)),
                          Element(D)),
                         lambda t: (0, W * jnp.maximum(t * (BQ // W) - 1, 0),
                                    0)),
            full((D, D)),
            full((D, D)),
            full((D, D)),
            full((D, D)),
            full((1, D)),
            full((1, D)),
            full((1, D)),
        ],
        out_specs=pl.BlockSpec((B, BQ, D), lambda t: (0, t, 0)),
        out_shape=jax.ShapeDtypeStruct((B, S, D), jnp.float32),
        scratch_shapes=[pltpu.VMEM((D, D), jnp.bfloat16)] * 4,
    )(hidden_states, hidden_states, w_qs, w_ks, w_vs, w_fc,
      b_fc.reshape(1, D), gamma.reshape(1, D), beta.reshape(1, D))
